# baseline (speedup 1.0000x reference)
import jax
import jax.numpy as jnp
from jax import lax
from jax.experimental import pallas as pl
from jax.experimental.pallas import tpu as pltpu
from jax.experimental.pallas import tpu_sc as plsc

_PC_START = (-54.0, -54.0)
_VOXEL = (0.075, 0.075)
_OUT_STRIDE = 8
_H = 180
_W = 180
_NPAD = 512
_NPTS = 5
_LANES = 16
_CHUNK = 16


def _bf16_round(v):
    u = plsc.bitcast(v, jnp.uint32)
    lsb = (u >> 16) & jnp.uint32(1)
    r = (u + jnp.uint32(0x7FFF) + lsb) & jnp.uint32(0xFFFF0000)
    return plsc.bitcast(r, jnp.float32)


def _sc_bev(table, rois_p, B, C):
    info = plsc.get_sparse_core_info()
    nc, ns = info.num_cores, info.num_subcores
    nw = nc * ns
    wpb = nw // B
    nspan = _NPAD // wpb
    ngrp = nspan // _LANES
    nchunk = _NPTS * (nspan // _CHUNK)
    inv_vox = 1.0 / (_VOXEL[0] * _OUT_STRIDE)

    mesh = plsc.VectorSubcoreMesh(core_axis_name="c", subcore_axis_name="s")

    @pl.kernel(
        out_type=jax.ShapeDtypeStruct((B, _NPAD, _NPTS * C), jnp.float32),
        mesh=mesh,
        compiler_params=pltpu.CompilerParams(needs_layout_passes=False),
        scratch_types=[
            pltpu.VMEM((8, _NPAD), jnp.float32),
            pltpu.VMEM((_NPTS * 4, 128), jnp.int32),
            pltpu.VMEM((4 * _NPTS * nspan,), jnp.float32),
            pltpu.VMEM((128, 128), jnp.float32),
            pltpu.VMEM((128, 128), jnp.float32),
            pltpu.VMEM((128, 128), jnp.float32),
            pltpu.VMEM((_CHUNK, C), jnp.float32),
            pltpu.VMEM((_CHUNK, C), jnp.float32),
            pltpu.VMEM((_CHUNK, C), jnp.float32),
            pltpu.SemaphoreType.DMA,
            pltpu.SemaphoreType.DMA,
            pltpu.SemaphoreType.DMA,
            pltpu.SemaphoreType.DMA,
            pltpu.SemaphoreType.DMA,
            pltpu.SemaphoreType.DMA,
        ],
    )
    def k(table_hbm, rois_hbm, out_hbm, rbuf, idx_buf, w_buf,
          rows0, rows1, rows2, ob0, ob1, ob2,
          sg0, sg1, sg2, so0, so1, so2):
        rows = (rows0, rows1, rows2)
        obs = (ob0, ob1, ob2)
        sgs = (sg0, sg1, sg2)
        sos = (so0, so1, so2)
        wid = lax.axis_index("s") * nc + lax.axis_index("c")
        b = wid // wpb
        nbase = (wid % wpb) * nspan
        for f in range(8):
            pltpu.sync_copy(rois_hbm.at[b * 8 + f], rbuf.at[f])

        for g in range(ngrp):
            sl = pl.ds(g * _LANES, _LANES)
            gsl = pl.ds(nbase + g * _LANES, _LANES)
            cx = rbuf[0, gsl]
            cy = rbuf[1, gsl]
            dx = rbuf[3, gsl]
            dy = rbuf[4, gsl]
            sb = _bf16_round(rbuf[5, gsl])
            cb = _bf16_round(rbuf[6, gsl])
            hx = _bf16_round(0.5 * dx)
            hy = _bf16_round(0.5 * dy)
            hxc = hx * cb
            hxs = hx * sb
            hyc = hy * cb
            hys = hy * sb
            pts = [
                (cx, cy),
                (cx - hxc, cy + hxs),
                (cx + hxc, cy - hxs),
                (cx - hys, cy - hyc),
                (cx + hys, cy + hyc),
            ]
            for pt, (xv, yv) in enumerate(pts):
                gx = (xv - _PC_START[0]) / _VOXEL[0] / _OUT_STRIDE
                gy = (yv - _PC_START[1]) / _VOXEL[1] / _OUT_STRIDE
                x0i = gx.astype(jnp.int32)
                y0i = gy.astype(jnp.int32)
                x0f = x0i.astype(jnp.float32)
                y0f = y0i.astype(jnp.float32)
                fx0 = x0f - gx
                fy0 = y0f - gy
                fx1 = fx0 + 1.0
                fy1 = fy0 + 1.0
                wbase = pt * nspan + g * _LANES
                w_buf[pl.ds(0 * _NPTS * nspan + wbase, _LANES)] = fx1 * fy1
                w_buf[pl.ds(1 * _NPTS * nspan + wbase, _LANES)] = -fx1 * fy0
                w_buf[pl.ds(2 * _NPTS * nspan + wbase, _LANES)] = -fx0 * fy1
                w_buf[pl.ds(3 * _NPTS * nspan + wbase, _LANES)] = fx0 * fy0
                r00 = (y0i * _W + x0i) * 8 + b
                for kt, off in enumerate(
                        (0, 4, 8 * _W, 8 * _W + 4, 8, 12, 8 * _W + 8, 8 * _W + 12)):
                    idx_buf[pt * 4 + g, pl.ds(kt * _LANES, _LANES)] = r00 + off

        pending_g = {}
        pending_o = {}
        nbuf = 3

        def fire(ci):
            pending_g[ci] = pltpu.async_copy(
                table_hbm.at[idx_buf.at[ci]], rows[ci % nbuf], sgs[ci % nbuf])

        def drain(ci):
            pending_g.pop(ci).wait()

        def out_slice(ci):
            pt, q = divmod(ci, 4)
            gn0 = nbase + q * _CHUNK
            return out_hbm.at[b, pl.ds(gn0, _CHUNK), pl.ds(pt * C, C)]

        def compute(ci):
            pt, q = divmod(ci, 4)
            par = ci % nbuf
            rr = rows[par]
            ob = obs[par]

            @plsc.parallel_loop(0, _CHUNK, unroll=4)
            def point(i):
                wb = pt * nspan + q * _CHUNK + i
                w0 = plsc.load_gather(
                    w_buf, [jnp.full((_LANES,), wb, jnp.int32)])
                w1 = plsc.load_gather(
                    w_buf, [jnp.full((_LANES,), wb + _NPTS * nspan, jnp.int32)])
                w2 = plsc.load_gather(
                    w_buf, [jnp.full((_LANES,), wb + 2 * _NPTS * nspan, jnp.int32)])
                w3 = plsc.load_gather(
                    w_buf, [jnp.full((_LANES,), wb + 3 * _NPTS * nspan, jnp.int32)])
                for t in range(2):
                    for l0 in range(128 // _LANES):
                        sl2 = pl.ds(l0 * _LANES, _LANES)
                        acc = rr[t * _LANES + i, sl2] * w0
                        acc = acc + rr[2 * _LANES + t * _LANES + i, sl2] * w1
                        acc = acc + rr[4 * _LANES + t * _LANES + i, sl2] * w2
                        acc = acc + rr[6 * _LANES + t * _LANES + i, sl2] * w3
                        ob[i, pl.ds(t * 128 + l0 * _LANES, _LANES)] = acc

        nchunkx = _NPTS * 4
        fire(0)
        fire(1)
        for ci in range(nchunkx):
            if ci + 2 < nchunkx:
                fire(ci + 2)
            drain(ci)
            if ci >= nbuf:
                pending_o.pop(ci - nbuf).wait()
            compute(ci)
            pending_o[ci] = pltpu.async_copy(
                obs[ci % nbuf], out_slice(ci), sos[ci % nbuf])
        for ci in range(nchunkx - nbuf, nchunkx):
            pending_o.pop(ci).wait()

    return k(table, rois_p)


def kernel(spatial_features_2d, rois):
    B, C, H, W = spatial_features_2d.shape
    N = rois.shape[1]
    table = (spatial_features_2d
             .transpose(2, 3, 1, 0)
             .reshape(H, W, 2, 128, B)
             .transpose(0, 1, 2, 4, 3)
             .reshape(H * W * 2 * B, 128))
    ang = rois[:, :, 6]
    rois_p = (jnp.zeros((B, 8, _NPAD), jnp.float32)
              .at[:, :5, :N].set(rois[:, :, :5].transpose(0, 2, 1))
              .at[:, 5, :N].set(jnp.sin(ang))
              .at[:, 6, :N].set(jnp.cos(ang))).reshape(B * 8, _NPAD)
    res = _sc_bev(table, rois_p, B, C)
    return res[:, :N]


# speedup vs baseline: 1.3058x; 1.3058x over previous
import jax
import jax.numpy as jnp
from jax import lax
from jax.experimental import pallas as pl
from jax.experimental.pallas import tpu as pltpu
from jax.experimental.pallas import tpu_sc as plsc

_PC_START = (-54.0, -54.0)
_VOXEL = (0.075, 0.075)
_OUT_STRIDE = 8
_H = 180
_W = 180
_NPAD = 512
_NPTS = 5
_LANES = 16
_CHUNK = 16


def _bf16_round(v):
    u = plsc.bitcast(v, jnp.uint32)
    lsb = (u >> 16) & jnp.uint32(1)
    r = (u + jnp.uint32(0x7FFF) + lsb) & jnp.uint32(0xFFFF0000)
    return plsc.bitcast(r, jnp.float32)


def _sc_bev(table, rois_p, B, C):
    info = plsc.get_sparse_core_info()
    nc, ns = info.num_cores, info.num_subcores
    nw = nc * ns
    wpb = nw // B
    nspan = _NPAD // wpb
    ngrp = nspan // _LANES
    nchunk = _NPTS * (nspan // _CHUNK)
    inv_vox = 1.0 / (_VOXEL[0] * _OUT_STRIDE)

    mesh = plsc.VectorSubcoreMesh(core_axis_name="c", subcore_axis_name="s")

    @pl.kernel(
        out_type=jax.ShapeDtypeStruct((B * _NPTS * _NPAD, C), jnp.float32),
        mesh=mesh,
        compiler_params=pltpu.CompilerParams(needs_layout_passes=False),
        scratch_types=[
            pltpu.VMEM((8, _NPAD), jnp.float32),
            pltpu.VMEM((_NPTS * 4, 128), jnp.int32),
            pltpu.VMEM((4 * _NPTS * nspan,), jnp.float32),
            pltpu.VMEM((128, 128), jnp.float32),
            pltpu.VMEM((128, 128), jnp.float32),
            pltpu.VMEM((128, 128), jnp.float32),
            pltpu.VMEM((_CHUNK, C), jnp.float32),
            pltpu.VMEM((_CHUNK, C), jnp.float32),
            pltpu.VMEM((_CHUNK, C), jnp.float32),
            pltpu.SemaphoreType.DMA,
            pltpu.SemaphoreType.DMA,
            pltpu.SemaphoreType.DMA,
            pltpu.SemaphoreType.DMA,
            pltpu.SemaphoreType.DMA,
            pltpu.SemaphoreType.DMA,
        ],
    )
    def k(table_hbm, rois_hbm, out_hbm, rbuf, idx_buf, w_buf,
          rows0, rows1, rows2, ob0, ob1, ob2,
          sg0, sg1, sg2, so0, so1, so2):
        rows = (rows0, rows1, rows2)
        obs = (ob0, ob1, ob2)
        sgs = (sg0, sg1, sg2)
        sos = (so0, so1, so2)
        wid = lax.axis_index("s") * nc + lax.axis_index("c")
        b = wid // wpb
        nbase = (wid % wpb) * nspan
        for f in range(8):
            pltpu.sync_copy(rois_hbm.at[b * 8 + f], rbuf.at[f])

        for g in range(ngrp):
            sl = pl.ds(g * _LANES, _LANES)
            gsl = pl.ds(nbase + g * _LANES, _LANES)
            cx = rbuf[0, gsl]
            cy = rbuf[1, gsl]
            dx = rbuf[3, gsl]
            dy = rbuf[4, gsl]
            sb = _bf16_round(rbuf[5, gsl])
            cb = _bf16_round(rbuf[6, gsl])
            hx = _bf16_round(0.5 * dx)
            hy = _bf16_round(0.5 * dy)
            hxc = hx * cb
            hxs = hx * sb
            hyc = hy * cb
            hys = hy * sb
            pts = [
                (cx, cy),
                (cx - hxc, cy + hxs),
                (cx + hxc, cy - hxs),
                (cx - hys, cy - hyc),
                (cx + hys, cy + hyc),
            ]
            for pt, (xv, yv) in enumerate(pts):
                gx = (xv - _PC_START[0]) / _VOXEL[0] / _OUT_STRIDE
                gy = (yv - _PC_START[1]) / _VOXEL[1] / _OUT_STRIDE
                x0i = gx.astype(jnp.int32)
                y0i = gy.astype(jnp.int32)
                x0f = x0i.astype(jnp.float32)
                y0f = y0i.astype(jnp.float32)
                fx0 = x0f - gx
                fy0 = y0f - gy
                fx1 = fx0 + 1.0
                fy1 = fy0 + 1.0
                wbase = pt * nspan + g * _LANES
                w_buf[pl.ds(0 * _NPTS * nspan + wbase, _LANES)] = fx1 * fy1
                w_buf[pl.ds(1 * _NPTS * nspan + wbase, _LANES)] = -fx1 * fy0
                w_buf[pl.ds(2 * _NPTS * nspan + wbase, _LANES)] = -fx0 * fy1
                w_buf[pl.ds(3 * _NPTS * nspan + wbase, _LANES)] = fx0 * fy0
                r00 = (y0i * _W + x0i) * 8 + b
                for kt, off in enumerate(
                        (0, 4, 8 * _W, 8 * _W + 4, 8, 12, 8 * _W + 8, 8 * _W + 12)):
                    idx_buf[pt * 4 + g, pl.ds(kt * _LANES, _LANES)] = r00 + off

        pending_g = {}
        pending_o = {}
        nbuf = 3

        def fire(ci):
            pending_g[ci] = pltpu.async_copy(
                table_hbm.at[idx_buf.at[ci]], rows[ci % nbuf], sgs[ci % nbuf])

        def drain(ci):
            pending_g.pop(ci).wait()

        def out_slice(ci):
            pt, q = divmod(ci, 4)
            gn0 = nbase + q * _CHUNK
            return out_hbm.at[pl.ds((b * _NPTS + pt) * _NPAD + gn0, _CHUNK)]

        def compute(ci):
            pt, q = divmod(ci, 4)
            par = ci % nbuf
            rr = rows[par]
            ob = obs[par]

            @plsc.parallel_loop(0, _CHUNK, unroll=4)
            def point(i):
                wb = pt * nspan + q * _CHUNK + i
                w0 = plsc.load_gather(
                    w_buf, [jnp.full((_LANES,), wb, jnp.int32)])
                w1 = plsc.load_gather(
                    w_buf, [jnp.full((_LANES,), wb + _NPTS * nspan, jnp.int32)])
                w2 = plsc.load_gather(
                    w_buf, [jnp.full((_LANES,), wb + 2 * _NPTS * nspan, jnp.int32)])
                w3 = plsc.load_gather(
                    w_buf, [jnp.full((_LANES,), wb + 3 * _NPTS * nspan, jnp.int32)])
                for t in range(2):
                    for l0 in range(128 // _LANES):
                        sl2 = pl.ds(l0 * _LANES, _LANES)
                        acc = rr[t * _LANES + i, sl2] * w0
                        acc = acc + rr[2 * _LANES + t * _LANES + i, sl2] * w1
                        acc = acc + rr[4 * _LANES + t * _LANES + i, sl2] * w2
                        acc = acc + rr[6 * _LANES + t * _LANES + i, sl2] * w3
                        ob[i, pl.ds(t * 128 + l0 * _LANES, _LANES)] = acc

        nchunkx = _NPTS * 4
        fire(0)
        fire(1)
        for ci in range(nchunkx):
            if ci + 2 < nchunkx:
                fire(ci + 2)
            drain(ci)
            if ci >= nbuf:
                pending_o.pop(ci - nbuf).wait()
            compute(ci)
            pending_o[ci] = pltpu.async_copy(
                obs[ci % nbuf], out_slice(ci), sos[ci % nbuf])
        for ci in range(nchunkx - nbuf, nchunkx):
            pending_o.pop(ci).wait()

    return k(table, rois_p)


def kernel(spatial_features_2d, rois):
    B, C, H, W = spatial_features_2d.shape
    N = rois.shape[1]
    table = (spatial_features_2d
             .transpose(2, 3, 1, 0)
             .reshape(H, W, 2, 128, B)
             .transpose(0, 1, 2, 4, 3)
             .reshape(H * W * 2 * B, 128))
    ang = rois[:, :, 6]
    rois_p = (jnp.zeros((B, 8, _NPAD), jnp.float32)
              .at[:, :5, :N].set(rois[:, :, :5].transpose(0, 2, 1))
              .at[:, 5, :N].set(jnp.sin(ang))
              .at[:, 6, :N].set(jnp.cos(ang))).reshape(B * 8, _NPAD)
    res = _sc_bev(table, rois_p, B, C).reshape(B, _NPTS, _NPAD, C)
    res = res[:, :, :N]
    return res.transpose(0, 2, 1, 3).reshape(B, N, _NPTS * C)
